# baseline (speedup 1.0000x reference)
import jax
import jax.numpy as jnp
from jax import lax
from jax.experimental import pallas as pl
from jax.experimental.pallas import tpu as pltpu

_TOPK = 100
_B = 4


def _topk_body(*refs):
    xr = refs[:_B]
    o_ref = refs[_B]
    _NEG = float("-inf")
    _BIG = 2**30
    f32 = jnp.float32
    a_io = lax.broadcasted_iota(jnp.int32, (80, 128), 0)
    b_io = lax.broadcasted_iota(jnp.int32, (80, 128), 1)
    ridx = a_io * 128 + b_io
    lane1 = lax.broadcasted_iota(jnp.int32, (1, 128), 1)

    ms = tuple(jnp.max(xr[i][0, :80, :, :], axis=2) for i in range(_B))

    def step(k, carry):
        ms, ois, oss = carry
        km = lane1 == k
        nm, noi, nos = [], [], []
        for i in range(_B):
            m = ms[i]
            gmax = jnp.max(m)
            rstar = jnp.min(jnp.where(m == gmax, ridx, _BIG))
            c = rstar >> 7
            y = rstar & 127
            row = xr[i][0, c, pl.ds(y, 1), :]
            lstar = jnp.min(jnp.where(row == gmax, lane1, _BIG))
            newrow = jnp.where(lane1 == lstar, _NEG, row)
            xr[i][0, c, pl.ds(y, 1), :] = newrow
            nm.append(jnp.where(ridx == rstar, jnp.max(newrow), m))
            noi.append(jnp.where(km, rstar * 128 + lstar, ois[i]))
            nos.append(jnp.where(km, gmax, oss[i]))
        return (tuple(nm), tuple(noi), tuple(nos))

    zi = tuple(jnp.zeros((1, 128), jnp.int32) for _ in range(_B))
    zs = tuple(jnp.zeros((1, 128), f32) for _ in range(_B))
    _, idxs, scores = lax.fori_loop(0, _TOPK, step, (ms, zi, zs))

    sub2d = lax.broadcasted_iota(jnp.int32, (128, 128), 0)
    dn = (((0,), (0,)), ((), ()))
    for i in range(_B):
        idx = idxs[i]
        y = (idx >> 7) & 127
        xl = idx & 127
        spat = idx & 16383
        by = (sub2d == y).astype(f32)
        bx = (sub2d == xl).astype(f32)

        def gather_ch(ch):
            p = lax.dot_general(
                ch, by, dn, preferred_element_type=f32,
                precision=lax.Precision.HIGHEST,
            )
            return jnp.sum(p * bx, axis=0, keepdims=True)

        bw = gather_ch(xr[i][0, 80, :, :])
        bh = gather_ch(xr[i][0, 81, :, :])
        xo = gather_ch(xr[i][0, 82, :, :])
        yo = gather_ch(xr[i][0, 83, :, :])
        cls = idx.astype(f32) / f32(16384.0)
        cy = spat.astype(f32) / f32(128.0) + yo
        cx = xl.astype(f32) + xo
        hw = 0.5 * bw
        hh = 0.5 * bh
        s4 = f32(4.0)
        o_ref[i] = jnp.concatenate(
            [(cx - hw) * s4, (cy - hh) * s4, (cx + hw) * s4, (cy + hh) * s4,
             cls, scores[i]],
            axis=0,
        )


def _make_spec(i):
    return pl.BlockSpec((1, 84, 128, 128), lambda b, i=i: (b * _B + i, 0, 0, 0))


def _build(interpret=False):
    return pl.pallas_call(
        _topk_body,
        grid=(16 // _B,),
        in_specs=[_make_spec(i) for i in range(_B)],
        out_specs=pl.BlockSpec((_B, 6, 128), lambda b: (b, 0, 0)),
        out_shape=jax.ShapeDtypeStruct((16, 6, 128), jnp.float32),
        interpret=interpret,
    )


@jax.jit
def kernel(x):
    rows = _build()(*([x] * _B))
    return jnp.transpose(rows, (0, 2, 1))[:, :_TOPK, :]


# speedup vs baseline: 1.0005x; 1.0005x over previous
import jax
import jax.numpy as jnp
from jax import lax
from jax.experimental import pallas as pl
from jax.experimental.pallas import tpu as pltpu

_TOPK = 100
_B = 4


def _topk_body(*refs):
    xr = refs[:_B]
    o_ref = refs[_B]
    _NEG = float("-inf")
    _BIG = 2**30
    f32 = jnp.float32
    a_io = lax.broadcasted_iota(jnp.int32, (80, 128), 0)
    b_io = lax.broadcasted_iota(jnp.int32, (80, 128), 1)
    ridx = a_io * 128 + b_io
    lane1 = lax.broadcasted_iota(jnp.int32, (1, 128), 1)

    ms = tuple(jnp.max(xr[i][0, :80, :, :], axis=2) for i in range(_B))

    def step(k, carry):
        ms, ois, oss = carry
        km = lane1 == k
        nm, noi, nos = [], [], []
        for i in range(_B):
            m = ms[i]
            gmaxv = jnp.max(m, axis=(0, 1), keepdims=True)
            rv = jnp.min(
                jnp.where(m == gmaxv, ridx, _BIG), axis=(0, 1), keepdims=True
            )
            rstar = rv[0, 0]
            c = rstar >> 7
            y = rstar & 127
            row = xr[i][0, c, pl.ds(y, 1), :]
            lstarv = jnp.min(
                jnp.where(row == gmaxv, lane1, _BIG), axis=(0, 1), keepdims=True
            )
            newrow = jnp.where(lane1 == lstarv, _NEG, row)
            xr[i][0, c, pl.ds(y, 1), :] = newrow
            newmaxv = jnp.max(newrow, axis=(0, 1), keepdims=True)
            nm.append(jnp.where(ridx == rv, newmaxv, m))
            noi.append(jnp.where(km, rv * 128 + lstarv, ois[i]))
            nos.append(jnp.where(km, gmaxv, oss[i]))
        return (tuple(nm), tuple(noi), tuple(nos))

    zi = tuple(jnp.zeros((1, 128), jnp.int32) for _ in range(_B))
    zs = tuple(jnp.zeros((1, 128), f32) for _ in range(_B))
    _, idxs, scores = lax.fori_loop(0, _TOPK, step, (ms, zi, zs))

    sub2d = lax.broadcasted_iota(jnp.int32, (128, 128), 0)
    dn = (((0,), (0,)), ((), ()))
    for i in range(_B):
        idx = idxs[i]
        y = (idx >> 7) & 127
        xl = idx & 127
        spat = idx & 16383
        by = (sub2d == y).astype(f32)
        bx = (sub2d == xl).astype(f32)

        def gather_ch(ch):
            p = lax.dot_general(
                ch, by, dn, preferred_element_type=f32,
                precision=lax.Precision.HIGHEST,
            )
            return jnp.sum(p * bx, axis=0, keepdims=True)

        bw = gather_ch(xr[i][0, 80, :, :])
        bh = gather_ch(xr[i][0, 81, :, :])
        xo = gather_ch(xr[i][0, 82, :, :])
        yo = gather_ch(xr[i][0, 83, :, :])
        cls = idx.astype(f32) / f32(16384.0)
        cy = spat.astype(f32) / f32(128.0) + yo
        cx = xl.astype(f32) + xo
        hw = 0.5 * bw
        hh = 0.5 * bh
        s4 = f32(4.0)
        o_ref[i] = jnp.concatenate(
            [(cx - hw) * s4, (cy - hh) * s4, (cx + hw) * s4, (cy + hh) * s4,
             cls, scores[i]],
            axis=0,
        )


def _make_spec(i):
    return pl.BlockSpec((1, 84, 128, 128), lambda b, i=i: (b * _B + i, 0, 0, 0))


def _build(interpret=False):
    return pl.pallas_call(
        _topk_body,
        grid=(16 // _B,),
        in_specs=[_make_spec(i) for i in range(_B)],
        out_specs=pl.BlockSpec((_B, 6, 128), lambda b: (b, 0, 0)),
        out_shape=jax.ShapeDtypeStruct((16, 6, 128), jnp.float32),
        interpret=interpret,
    )


@jax.jit
def kernel(x):
    rows = _build()(*([x] * _B))
    return jnp.transpose(rows, (0, 2, 1))[:, :_TOPK, :]
